# baseline (speedup 1.0000x reference)
import functools

import jax
import jax.numpy as jnp
from jax import lax
from jax.experimental import pallas as pl
from jax.experimental.pallas import tpu as pltpu
from jax.experimental.pallas import tpu_sc as plsc

H = 16
DH = 64
KS = 4
TOPK = 64
SCALE = 1.0 / (DH ** 0.5)

TS = 512

def _qkv_pool_body(hs_ref, wq_ref, wk_ref, wv_ref, q_ref, pk_ref, pv_ref):
    hs = hs_ref[0]
    q_ref[0] = jnp.dot(hs, wq_ref[...], preferred_element_type=jnp.float32)
    ts, d = hs.shape
    hsp = hs.reshape(ts // KS, KS, d).mean(axis=1)
    pk_ref[0] = jnp.dot(hsp, wk_ref[...], preferred_element_type=jnp.float32)
    pv_ref[0] = jnp.dot(hsp, wv_ref[...], preferred_element_type=jnp.float32)


NCAND = 96
NCHUNKS = 1024


def _routing_screen_body(q_ref, pk_ref, cand_ref, rows_ref, macc_ref):
    b = pl.program_id(0)
    st = pl.program_id(1)
    n_st = pl.num_programs(1)
    q = q_ref[0]
    k = pk_ref[0]
    m = None
    for h in range(H):
        qh = q[:, h * DH:(h + 1) * DH]
        kh = k[:, h * DH:(h + 1) * DH]
        s = lax.dot_general(qh, kh, (((1,), (1,)), ((), ())),
                            preferred_element_type=jnp.float32)
        mh = jnp.max(s, axis=0, keepdims=True)
        m = mh if m is None else jnp.maximum(m, mh)

    @pl.when(st == 0)
    def _init():
        macc_ref[...] = m

    @pl.when(st > 0)
    def _acc():
        macc_ref[...] = jnp.maximum(macc_ref[...], m)

    @pl.when(st == n_st - 1)
    def _topk():
        sc0 = macc_ref[...]
        c = sc0.shape[1]
        iota_c = lax.broadcasted_iota(jnp.int32, (1, c), 1)
        iota_k = lax.broadcasted_iota(jnp.int32, (1, NCAND), 1)

        def body(i, carry):
            sc, acc = carry
            mx = jnp.max(sc)
            sel = jnp.min(jnp.where(sc == mx, iota_c, jnp.int32(c)))
            acc = jnp.where(iota_k == i, sel, acc)
            sc = jnp.where(iota_c == sel, -jnp.inf, sc)
            return sc, acc

        _, acc = lax.fori_loop(0, NCAND, body,
                               (sc0, jnp.zeros((1, NCAND), jnp.int32)))
        cand_ref[0] = acc
        rows_ref[0] = jnp.concatenate(
            [acc * KS + j for j in range(KS)], axis=1) + b * (c * KS)


def _rescue_body(q_ref, hsrows_ref, wk_ref, cand_ref, idx_ref,
                 pkc_ref, macc_ref):
    b = pl.program_id(0)
    st = pl.program_id(1)
    n_st = pl.num_programs(1)

    @pl.when(st == 0)
    def _proj():
        rows = hsrows_ref[0]
        krows = jnp.dot(rows, wk_ref[...],
                        preferred_element_type=jnp.float32)
        pkc_ref[...] = krows.reshape(KS, NCAND, krows.shape[1]).mean(axis=0)

    q = q_ref[0]
    kc = pkc_ref[...]
    m = None
    for h in range(H):
        qh = q[:, h * DH:(h + 1) * DH]
        kh = kc[:, h * DH:(h + 1) * DH]
        s = lax.dot_general(qh, kh, (((1,), (1,)), ((), ())),
                            preferred_element_type=jnp.float32)
        mh = jnp.max(s, axis=0, keepdims=True)
        m = mh if m is None else jnp.maximum(m, mh)

    @pl.when(st == 0)
    def _init():
        macc_ref[...] = m

    @pl.when(st > 0)
    def _acc():
        macc_ref[...] = jnp.maximum(macc_ref[...], m)

    @pl.when(st == n_st - 1)
    def _topk():
        sc0 = macc_ref[...]
        cand = cand_ref[0]
        iota_n = lax.broadcasted_iota(jnp.int32, (1, NCAND), 1)
        iota_k = lax.broadcasted_iota(jnp.int32, (1, TOPK), 1)

        def body(i, carry):
            sc, acc = carry
            mx = jnp.max(sc)
            sel = jnp.min(jnp.where(sc == mx, iota_n, jnp.int32(NCAND)))
            gid = jnp.sum(jnp.where(iota_n == sel, cand, 0))
            acc = jnp.where(iota_k == i, gid, acc)
            sc = jnp.where(iota_n == sel, -jnp.inf, sc)
            return sc, acc

        _, acc = lax.fori_loop(0, TOPK, body,
                               (sc0, jnp.zeros((1, TOPK), jnp.int32)))
        idx_ref[0] = acc + b * NCHUNKS


def _attn_body(q_ref, sk_ref, sv_ref, wo_ref, o_ref):
    q = q_ref[0].astype(jnp.bfloat16)
    sk = sk_ref[0].astype(jnp.bfloat16)
    sv = sv_ref[0].astype(jnp.bfloat16)
    outs = []
    for h in range(H):
        qh = q[:, h * DH:(h + 1) * DH]
        kh = sk[:, h * DH:(h + 1) * DH]
        vh = sv[:, h * DH:(h + 1) * DH]
        logits = lax.dot_general(qh, kh, (((1,), (1,)), ((), ())),
                                 preferred_element_type=jnp.float32) * SCALE
        m = jnp.max(logits, axis=1, keepdims=True)
        p = jnp.exp(logits - m)
        attn = p / jnp.sum(p, axis=1, keepdims=True)
        outs.append(jnp.dot(attn.astype(jnp.bfloat16), vh,
                            preferred_element_type=jnp.float32))
    o = jnp.concatenate(outs, axis=1).astype(jnp.bfloat16)
    o_ref[0] = jnp.dot(o, wo_ref[...], preferred_element_type=jnp.float32)


def _sc_gather(pk_flat, pv_flat, idx_flat):
    n_rows = idx_flat.shape[0]
    d = pk_flat.shape[1]
    rows_per_w = 8
    n_workers = n_rows // rows_per_w

    @functools.partial(
        pl.kernel,
        mesh=plsc.VectorSubcoreMesh(core_axis_name="c", subcore_axis_name="s"),
        out_type=[jax.ShapeDtypeStruct((n_rows, d), jnp.float32),
                  jax.ShapeDtypeStruct((n_rows, d), jnp.float32)],
        scratch_types=[pltpu.VMEM((rows_per_w,), jnp.int32),
                       pltpu.VMEM((rows_per_w, d), jnp.float32),
                       pltpu.VMEM((rows_per_w, d), jnp.float32),
                       pltpu.SemaphoreType.DMA],
    )
    def gather(pk_hbm, pv_hbm, idx_hbm, selk_hbm, selv_hbm,
               idx_v, krows_v, vrows_v, sem):
        wid = lax.axis_index("s") * 2 + lax.axis_index("c")

        @pl.when(wid < n_workers)
        def _():
            base = wid * rows_per_w
            pltpu.sync_copy(idx_hbm.at[pl.ds(base, rows_per_w)], idx_v)
            pltpu.async_copy(pk_hbm.at[idx_v], krows_v, sem).wait()
            pltpu.sync_copy(krows_v, selk_hbm.at[pl.ds(base, rows_per_w)])
            pltpu.async_copy(pv_hbm.at[idx_v], vrows_v, sem).wait()
            pltpu.sync_copy(vrows_v, selv_hbm.at[pl.ds(base, rows_per_w)])

    return gather(pk_flat, pv_flat, idx_flat)


def _sc_gather_rows(hs_flat, idx_flat):
    n_rows = idx_flat.shape[0]
    d = hs_flat.shape[1]
    rows_per_w = n_rows // 32

    @functools.partial(
        pl.kernel,
        mesh=plsc.VectorSubcoreMesh(core_axis_name="c", subcore_axis_name="s"),
        out_type=jax.ShapeDtypeStruct((n_rows, d), jnp.float32),
        scratch_types=[pltpu.VMEM((rows_per_w,), jnp.int32),
                       pltpu.VMEM((rows_per_w, d), jnp.float32),
                       pltpu.SemaphoreType.DMA],
    )
    def gather(hs_hbm, idx_hbm, out_hbm, idx_v, rows_v, sem):
        wid = lax.axis_index("s") * 2 + lax.axis_index("c")
        base = wid * rows_per_w
        pltpu.sync_copy(idx_hbm.at[pl.ds(base, rows_per_w)], idx_v)
        pltpu.async_copy(hs_hbm.at[idx_v], rows_v, sem).wait()
        pltpu.sync_copy(rows_v, out_hbm.at[pl.ds(base, rows_per_w)])

    return gather(hs_flat, idx_flat)


def kernel(hidden_states, Wq, Wk, Wv, Wo):
    B, S, D = hidden_states.shape
    C = S // KS
    n_tiles = S // TS

    q, pk, pv = pl.pallas_call(
        _qkv_pool_body,
        grid=(B, n_tiles),
        in_specs=[
            pl.BlockSpec((1, TS, D), lambda b, s: (b, s, 0)),
            pl.BlockSpec((D, D), lambda b, s: (0, 0)),
            pl.BlockSpec((D, D), lambda b, s: (0, 0)),
            pl.BlockSpec((D, D), lambda b, s: (0, 0)),
        ],
        out_specs=[
            pl.BlockSpec((1, TS, D), lambda b, s: (b, s, 0)),
            pl.BlockSpec((1, TS // KS, D), lambda b, s: (b, s, 0)),
            pl.BlockSpec((1, TS // KS, D), lambda b, s: (b, s, 0)),
        ],
        out_shape=[
            jax.ShapeDtypeStruct((B, S, D), jnp.float32),
            jax.ShapeDtypeStruct((B, C, D), jnp.float32),
            jax.ShapeDtypeStruct((B, C, D), jnp.float32),
        ],
    )(hidden_states, Wq, Wk, Wv)

    cand, rows = pl.pallas_call(
        _routing_screen_body,
        grid=(B, n_tiles),
        in_specs=[
            pl.BlockSpec((1, TS, D), lambda b, s: (b, s, 0)),
            pl.BlockSpec((1, C, D), lambda b, s: (b, 0, 0)),
        ],
        out_specs=[
            pl.BlockSpec((1, 1, NCAND), lambda b, s: (b, 0, 0)),
            pl.BlockSpec((1, 1, KS * NCAND), lambda b, s: (b, 0, 0)),
        ],
        out_shape=[
            jax.ShapeDtypeStruct((B, 1, NCAND), jnp.int32),
            jax.ShapeDtypeStruct((B, 1, KS * NCAND), jnp.int32),
        ],
        scratch_shapes=[pltpu.VMEM((1, C), jnp.float32)],
    )(q, pk)

    hs_rows = _sc_gather_rows(hidden_states.reshape(B * S, D),
                              rows.reshape(B * KS * NCAND))

    idx = pl.pallas_call(
        _rescue_body,
        grid=(B, n_tiles),
        in_specs=[
            pl.BlockSpec((1, TS, D), lambda b, s: (b, s, 0)),
            pl.BlockSpec((1, KS * NCAND, D), lambda b, s: (b, 0, 0)),
            pl.BlockSpec((D, D), lambda b, s: (0, 0)),
            pl.BlockSpec((1, 1, NCAND), lambda b, s: (b, 0, 0)),
        ],
        out_specs=pl.BlockSpec((1, 1, TOPK), lambda b, s: (b, 0, 0)),
        out_shape=jax.ShapeDtypeStruct((B, 1, TOPK), jnp.int32),
        scratch_shapes=[pltpu.VMEM((NCAND, D), jnp.float32),
                        pltpu.VMEM((1, NCAND), jnp.float32)],
    )(q, hs_rows.reshape(B, KS * NCAND, D), Wk, cand)

    selk, selv = _sc_gather(pk.reshape(B * C, D), pv.reshape(B * C, D),
                            idx.reshape(B * TOPK))

    out = pl.pallas_call(
        _attn_body,
        grid=(B, n_tiles),
        in_specs=[
            pl.BlockSpec((1, TS, D), lambda b, s: (b, s, 0)),
            pl.BlockSpec((1, TOPK, D), lambda b, s: (b, 0, 0)),
            pl.BlockSpec((1, TOPK, D), lambda b, s: (b, 0, 0)),
            pl.BlockSpec((D, D), lambda b, s: (0, 0)),
        ],
        out_specs=pl.BlockSpec((1, TS, D), lambda b, s: (b, s, 0)),
        out_shape=jax.ShapeDtypeStruct((B, S, D), jnp.float32),
    )(q, selk.reshape(B, TOPK, D), selv.reshape(B, TOPK, D),
      Wo.astype(jnp.bfloat16))

    return out


# speedup vs baseline: 1.1935x; 1.1935x over previous
import functools

import jax
import jax.numpy as jnp
from jax import lax
from jax.experimental import pallas as pl
from jax.experimental.pallas import tpu as pltpu
from jax.experimental.pallas import tpu_sc as plsc

H = 16
DH = 64
KS = 4
TOPK = 64
SCALE = 1.0 / (DH ** 0.5)

TS = 512

def _qkv_pool_body(hs_ref, wq_ref, wk_ref, wv_ref, q_ref, pk_ref, pv_ref):
    hs = hs_ref[0]
    q_ref[0] = jnp.dot(hs, wq_ref[...], preferred_element_type=jnp.float32)
    k = jnp.dot(hs, wk_ref[...], preferred_element_type=jnp.float32)
    ts, d = hs.shape
    pk_ref[0] = k.reshape(ts // KS, KS, d).mean(axis=1)
    hsp = hs.reshape(ts // KS, KS, d).mean(axis=1)
    pv_ref[0] = jnp.dot(hsp, wv_ref[...], preferred_element_type=jnp.float32)


def _routing_topk_body(q_ref, pk_ref, idx_ref, macc_ref):
    b = pl.program_id(0)
    st = pl.program_id(1)
    n_st = pl.num_programs(1)
    q = q_ref[0]
    k = pk_ref[0]
    m = None
    for h in range(H):
        qh = q[:, h * DH:(h + 1) * DH]
        kh = k[:, h * DH:(h + 1) * DH]
        s = lax.dot_general(qh, kh, (((1,), (1,)), ((), ())),
                            preferred_element_type=jnp.float32)
        mh = jnp.max(s, axis=0, keepdims=True)
        m = mh if m is None else jnp.maximum(m, mh)

    @pl.when(st == 0)
    def _init():
        macc_ref[...] = m

    @pl.when(st > 0)
    def _acc():
        macc_ref[...] = jnp.maximum(macc_ref[...], m)

    @pl.when(st == n_st - 1)
    def _topk():
        sc0 = macc_ref[...]
        c = sc0.shape[1]
        iota_c = lax.broadcasted_iota(jnp.int32, (1, c), 1)
        iota_k = lax.broadcasted_iota(jnp.int32, (1, TOPK), 1)

        def body(i, carry):
            sc, acc = carry
            mx = jnp.max(sc)
            sel = jnp.min(jnp.where(sc == mx, iota_c, jnp.int32(c)))
            acc = jnp.where(iota_k == i, sel, acc)
            sc = jnp.where(iota_c == sel, -jnp.inf, sc)
            return sc, acc

        _, acc = lax.fori_loop(0, TOPK, body,
                               (sc0, jnp.zeros((1, TOPK), jnp.int32)))
        idx_ref[0] = acc + b * c


def _attn_body(q_ref, sk_ref, sv_ref, wo_ref, o_ref):
    q = q_ref[0].astype(jnp.bfloat16)
    sk = sk_ref[0].astype(jnp.bfloat16)
    sv = sv_ref[0].astype(jnp.bfloat16)
    outs = []
    for h in range(H):
        qh = q[:, h * DH:(h + 1) * DH]
        kh = sk[:, h * DH:(h + 1) * DH]
        vh = sv[:, h * DH:(h + 1) * DH]
        logits = lax.dot_general(qh, kh, (((1,), (1,)), ((), ())),
                                 preferred_element_type=jnp.float32) * SCALE
        m = jnp.max(logits, axis=1, keepdims=True)
        p = jnp.exp(logits - m)
        attn = p / jnp.sum(p, axis=1, keepdims=True)
        outs.append(jnp.dot(attn.astype(jnp.bfloat16), vh,
                            preferred_element_type=jnp.float32))
    o = jnp.concatenate(outs, axis=1).astype(jnp.bfloat16)
    o_ref[0] = jnp.dot(o, wo_ref[...], preferred_element_type=jnp.float32)


def _sc_gather(pk_flat, pv_flat, idx_flat):
    n_rows = idx_flat.shape[0]
    d = pk_flat.shape[1]
    rows_per_w = 8
    n_workers = n_rows // rows_per_w

    @functools.partial(
        pl.kernel,
        mesh=plsc.VectorSubcoreMesh(core_axis_name="c", subcore_axis_name="s"),
        out_type=[jax.ShapeDtypeStruct((n_rows, d), jnp.float32),
                  jax.ShapeDtypeStruct((n_rows, d), jnp.float32)],
        scratch_types=[pltpu.VMEM((rows_per_w,), jnp.int32),
                       pltpu.VMEM((rows_per_w, d), jnp.float32),
                       pltpu.VMEM((rows_per_w, d), jnp.float32),
                       pltpu.SemaphoreType.DMA],
    )
    def gather(pk_hbm, pv_hbm, idx_hbm, selk_hbm, selv_hbm,
               idx_v, krows_v, vrows_v, sem):
        wid = lax.axis_index("s") * 2 + lax.axis_index("c")

        @pl.when(wid < n_workers)
        def _():
            base = wid * rows_per_w
            pltpu.sync_copy(idx_hbm.at[pl.ds(base, rows_per_w)], idx_v)
            pltpu.async_copy(pk_hbm.at[idx_v], krows_v, sem).wait()
            pltpu.sync_copy(krows_v, selk_hbm.at[pl.ds(base, rows_per_w)])
            pltpu.async_copy(pv_hbm.at[idx_v], vrows_v, sem).wait()
            pltpu.sync_copy(vrows_v, selv_hbm.at[pl.ds(base, rows_per_w)])

    return gather(pk_flat, pv_flat, idx_flat)


def kernel(hidden_states, Wq, Wk, Wv, Wo):
    B, S, D = hidden_states.shape
    C = S // KS
    n_tiles = S // TS

    q, pk, pv = pl.pallas_call(
        _qkv_pool_body,
        grid=(B, n_tiles),
        in_specs=[
            pl.BlockSpec((1, TS, D), lambda b, s: (b, s, 0)),
            pl.BlockSpec((D, D), lambda b, s: (0, 0)),
            pl.BlockSpec((D, D), lambda b, s: (0, 0)),
            pl.BlockSpec((D, D), lambda b, s: (0, 0)),
        ],
        out_specs=[
            pl.BlockSpec((1, TS, D), lambda b, s: (b, s, 0)),
            pl.BlockSpec((1, TS // KS, D), lambda b, s: (b, s, 0)),
            pl.BlockSpec((1, TS // KS, D), lambda b, s: (b, s, 0)),
        ],
        out_shape=[
            jax.ShapeDtypeStruct((B, S, D), jnp.float32),
            jax.ShapeDtypeStruct((B, C, D), jnp.float32),
            jax.ShapeDtypeStruct((B, C, D), jnp.float32),
        ],
    )(hidden_states, Wq, Wk, Wv)

    idx = pl.pallas_call(
        _routing_topk_body,
        grid=(B, n_tiles),
        in_specs=[
            pl.BlockSpec((1, TS, D), lambda b, s: (b, s, 0)),
            pl.BlockSpec((1, C, D), lambda b, s: (b, 0, 0)),
        ],
        out_specs=pl.BlockSpec((1, 1, TOPK), lambda b, s: (b, 0, 0)),
        out_shape=jax.ShapeDtypeStruct((B, 1, TOPK), jnp.int32),
        scratch_shapes=[pltpu.VMEM((1, C), jnp.float32)],
    )(q, pk)

    selk, selv = _sc_gather(pk.reshape(B * C, D), pv.reshape(B * C, D),
                            idx.reshape(B * TOPK))

    out = pl.pallas_call(
        _attn_body,
        grid=(B, n_tiles),
        in_specs=[
            pl.BlockSpec((1, TS, D), lambda b, s: (b, s, 0)),
            pl.BlockSpec((1, TOPK, D), lambda b, s: (b, 0, 0)),
            pl.BlockSpec((1, TOPK, D), lambda b, s: (b, 0, 0)),
            pl.BlockSpec((D, D), lambda b, s: (0, 0)),
        ],
        out_specs=pl.BlockSpec((1, TS, D), lambda b, s: (b, s, 0)),
        out_shape=jax.ShapeDtypeStruct((B, S, D), jnp.float32),
    )(q, selk.reshape(B, TOPK, D), selv.reshape(B, TOPK, D),
      Wo.astype(jnp.bfloat16))

    return out


# speedup vs baseline: 1.2417x; 1.0404x over previous
import functools

import jax
import jax.numpy as jnp
from jax import lax
from jax.experimental import pallas as pl
from jax.experimental.pallas import tpu as pltpu
from jax.experimental.pallas import tpu_sc as plsc

H = 16
DH = 64
KS = 4
TOPK = 64
SCALE = 1.0 / (DH ** 0.5)

TS = 1024

def _qkv_pool_body(hs_ref, wq_ref, wk_ref, wv_ref, q_ref, pk_ref, pv_ref):
    hs = hs_ref[0]
    q_ref[0] = jnp.dot(hs, wq_ref[...], preferred_element_type=jnp.float32)
    k = jnp.dot(hs, wk_ref[...], preferred_element_type=jnp.float32)
    ts, d = hs.shape
    pk_ref[0] = k.reshape(ts // KS, KS, d).mean(axis=1)
    hsp = hs.reshape(ts // KS, KS, d).mean(axis=1)
    pv_ref[0] = jnp.dot(hsp, wv_ref[...], preferred_element_type=jnp.float32)


def _routing_topk_body(q_ref, pk_ref, idx_ref, macc_ref):
    b = pl.program_id(0)
    st = pl.program_id(1)
    n_st = pl.num_programs(1)
    q = q_ref[0]
    k = pk_ref[0]
    m = None
    for h in range(H):
        qh = q[:, h * DH:(h + 1) * DH]
        kh = k[:, h * DH:(h + 1) * DH]
        s = lax.dot_general(qh, kh, (((1,), (1,)), ((), ())),
                            preferred_element_type=jnp.float32)
        mh = jnp.max(s, axis=0, keepdims=True)
        m = mh if m is None else jnp.maximum(m, mh)

    @pl.when(st == 0)
    def _init():
        macc_ref[...] = m

    @pl.when(st > 0)
    def _acc():
        macc_ref[...] = jnp.maximum(macc_ref[...], m)

    @pl.when(st == n_st - 1)
    def _topk():
        sc0 = macc_ref[...]
        c = sc0.shape[1]
        iota_c = lax.broadcasted_iota(jnp.int32, (1, c), 1)
        iota_k = lax.broadcasted_iota(jnp.int32, (1, TOPK), 1)

        def body(i, carry):
            sc, acc = carry
            mx = jnp.max(sc)
            sel = jnp.min(jnp.where(sc == mx, iota_c, jnp.int32(c)))
            acc = jnp.where(iota_k == i, sel, acc)
            sc = jnp.where(iota_c == sel, -jnp.inf, sc)
            return sc, acc

        _, acc = lax.fori_loop(0, TOPK, body,
                               (sc0, jnp.zeros((1, TOPK), jnp.int32)))
        idx_ref[0] = acc + b * c


def _attn_body(q_ref, sk_ref, sv_ref, wo_ref, o_ref):
    q = q_ref[0].astype(jnp.bfloat16)
    sk = sk_ref[0].astype(jnp.bfloat16)
    sv = sv_ref[0].astype(jnp.bfloat16)
    outs = []
    for h in range(H):
        qh = q[:, h * DH:(h + 1) * DH]
        kh = sk[:, h * DH:(h + 1) * DH]
        vh = sv[:, h * DH:(h + 1) * DH]
        logits = lax.dot_general(qh, kh, (((1,), (1,)), ((), ())),
                                 preferred_element_type=jnp.float32) * SCALE
        m = jnp.max(logits, axis=1, keepdims=True)
        p = jnp.exp(logits - m)
        attn = p / jnp.sum(p, axis=1, keepdims=True)
        outs.append(jnp.dot(attn.astype(jnp.bfloat16), vh,
                            preferred_element_type=jnp.float32))
    o = jnp.concatenate(outs, axis=1).astype(jnp.bfloat16)
    o_ref[0] = jnp.dot(o, wo_ref[...], preferred_element_type=jnp.float32)


def _sc_gather(pk_flat, pv_flat, idx_flat):
    n_rows = idx_flat.shape[0]
    d = pk_flat.shape[1]
    rows_per_w = 8
    n_workers = n_rows // rows_per_w

    @functools.partial(
        pl.kernel,
        mesh=plsc.VectorSubcoreMesh(core_axis_name="c", subcore_axis_name="s"),
        out_type=[jax.ShapeDtypeStruct((n_rows, d), jnp.float32),
                  jax.ShapeDtypeStruct((n_rows, d), jnp.float32)],
        scratch_types=[pltpu.VMEM((rows_per_w,), jnp.int32),
                       pltpu.VMEM((rows_per_w, d), jnp.float32),
                       pltpu.VMEM((rows_per_w, d), jnp.float32),
                       pltpu.SemaphoreType.DMA],
    )
    def gather(pk_hbm, pv_hbm, idx_hbm, selk_hbm, selv_hbm,
               idx_v, krows_v, vrows_v, sem):
        wid = lax.axis_index("s") * 2 + lax.axis_index("c")

        @pl.when(wid < n_workers)
        def _():
            base = wid * rows_per_w
            pltpu.sync_copy(idx_hbm.at[pl.ds(base, rows_per_w)], idx_v)
            pltpu.async_copy(pk_hbm.at[idx_v], krows_v, sem).wait()
            pltpu.sync_copy(krows_v, selk_hbm.at[pl.ds(base, rows_per_w)])
            pltpu.async_copy(pv_hbm.at[idx_v], vrows_v, sem).wait()
            pltpu.sync_copy(vrows_v, selv_hbm.at[pl.ds(base, rows_per_w)])

    return gather(pk_flat, pv_flat, idx_flat)


def kernel(hidden_states, Wq, Wk, Wv, Wo):
    B, S, D = hidden_states.shape
    C = S // KS
    n_tiles = S // TS

    q, pk, pv = pl.pallas_call(
        _qkv_pool_body,
        grid=(B, n_tiles),
        in_specs=[
            pl.BlockSpec((1, TS, D), lambda b, s: (b, s, 0)),
            pl.BlockSpec((D, D), lambda b, s: (0, 0)),
            pl.BlockSpec((D, D), lambda b, s: (0, 0)),
            pl.BlockSpec((D, D), lambda b, s: (0, 0)),
        ],
        out_specs=[
            pl.BlockSpec((1, TS, D), lambda b, s: (b, s, 0)),
            pl.BlockSpec((1, TS // KS, D), lambda b, s: (b, s, 0)),
            pl.BlockSpec((1, TS // KS, D), lambda b, s: (b, s, 0)),
        ],
        out_shape=[
            jax.ShapeDtypeStruct((B, S, D), jnp.float32),
            jax.ShapeDtypeStruct((B, C, D), jnp.float32),
            jax.ShapeDtypeStruct((B, C, D), jnp.float32),
        ],
    )(hidden_states, Wq, Wk, Wv)

    idx = pl.pallas_call(
        _routing_topk_body,
        grid=(B, n_tiles),
        in_specs=[
            pl.BlockSpec((1, TS, D), lambda b, s: (b, s, 0)),
            pl.BlockSpec((1, C, D), lambda b, s: (b, 0, 0)),
        ],
        out_specs=pl.BlockSpec((1, 1, TOPK), lambda b, s: (b, 0, 0)),
        out_shape=jax.ShapeDtypeStruct((B, 1, TOPK), jnp.int32),
        scratch_shapes=[pltpu.VMEM((1, C), jnp.float32)],
    )(q, pk)

    selk, selv = _sc_gather(pk.reshape(B * C, D), pv.reshape(B * C, D),
                            idx.reshape(B * TOPK))

    out = pl.pallas_call(
        _attn_body,
        grid=(B, n_tiles),
        in_specs=[
            pl.BlockSpec((1, TS, D), lambda b, s: (b, s, 0)),
            pl.BlockSpec((1, TOPK, D), lambda b, s: (b, 0, 0)),
            pl.BlockSpec((1, TOPK, D), lambda b, s: (b, 0, 0)),
            pl.BlockSpec((D, D), lambda b, s: (0, 0)),
        ],
        out_specs=pl.BlockSpec((1, TS, D), lambda b, s: (b, s, 0)),
        out_shape=jax.ShapeDtypeStruct((B, S, D), jnp.float32),
    )(q, selk.reshape(B, TOPK, D), selv.reshape(B, TOPK, D),
      Wo.astype(jnp.bfloat16))

    return out
